# Initial kernel scaffold; baseline (speedup 1.0000x reference)
import jax
import jax.numpy as jnp
from jax.experimental import pallas as pl


def kernel(neighbor_index, vertices, feature_map, directions, W, b, gamma, beta):
    raise NotImplementedError("write your pallas kernel here")


# speedup vs baseline: 9.9009x; 9.9009x over previous
import functools

import jax
import jax.numpy as jnp
from jax import lax
from jax.experimental import pallas as pl
from jax.experimental.pallas import tpu as pltpu
from jax.experimental.pallas import tpu_sc as plsc

_NC = 2
_NS = 16
_L = 16


def _rsqrt16(nsq):
    i = plsc.bitcast(nsq, jnp.int32)
    i = jnp.int32(0x5F3759DF) - lax.shift_right_logical(i, 1)
    r = plsc.bitcast(i, jnp.float32)
    half = nsq * 0.5
    for _ in range(3):
        r = r * (1.5 - half * r * r)
    return jnp.minimum(r, 1e12)


def _splat(ref, idxs):
    return plsc.load_gather(
        ref, [jnp.full((_L,), i, jnp.int32) for i in idxs])


def _make_prep(BS, V, VP, N, CHN, CHW):
    mesh = plsc.VectorSubcoreMesh(core_axis_name="c", subcore_axis_name="s")

    @functools.partial(
        pl.kernel,
        out_type=jax.ShapeDtypeStruct((BS * VP,), jnp.float32),
        mesh=mesh,
        compiler_params=pltpu.CompilerParams(needs_layout_passes=False),
        scratch_types=(
            pltpu.VMEM((CHN, CHW), jnp.int32),
            pltpu.VMEM((CHW,), jnp.float32),
            pltpu.VMEM_SHARED((VP,), jnp.float32),
            pltpu.SemaphoreType.DMA,
        ),
    )
    def prep(idx_hbm, zeros_hbm, ones_hbm, cnt_hbm,
             idx_v, ones_v, hist_sh, sem):
        c = lax.axis_index("c")
        s = lax.axis_index("s")
        trow = (c * _NS + s) * CHN
        pltpu.sync_copy(idx_hbm.at[pl.ds(trow, CHN)], idx_v)
        pltpu.sync_copy(ones_hbm, ones_v)

        @pl.when(s == 0)
        def _():
            pltpu.sync_copy(zeros_hbm, hist_sh)

        plsc.subcore_barrier()

        def chunk_body(j, carry):
            pltpu.sync_copy(ones_v, hist_sh.at[idx_v.at[j]], add=True)
            return carry

        lax.fori_loop(0, CHN, chunk_body, 0)

        plsc.subcore_barrier()

        @pl.when(s == 0)
        def _():
            pltpu.sync_copy(hist_sh, cnt_hbm.at[pl.ds(c * VP, VP)])

    return prep


def _make_fuse(BS, V, VP, N, OC, CHN, CHW):
    VPC = CHW // N
    VPT = VP // _NS
    CG = OC // _L
    VH = V // 2
    ZR = (VH // _NS) // 8 * 8
    VZ = VH + 8
    mesh = plsc.VectorSubcoreMesh(core_axis_name="c", subcore_axis_name="s")

    @functools.partial(
        pl.kernel,
        out_type=jax.ShapeDtypeStruct((BS * VP, OC), jnp.float32),
        mesh=mesh,
        compiler_params=pltpu.CompilerParams(needs_layout_passes=False),
        scratch_types=(
            pltpu.VMEM((CHN, CHW), jnp.int32),
            pltpu.VMEM((3 * VP,), jnp.float32),
            pltpu.VMEM((8, OC), jnp.float32),
        ) + (pltpu.VMEM((CHN, CHW), jnp.int32),)
          + tuple(pltpu.VMEM((CHW, OC), jnp.int32) for _ in range(2))
          + tuple(pltpu.VMEM((VPC, OC), jnp.float32) for _ in range(2))
          + (pltpu.VMEM_SHARED((VZ, OC), jnp.int32),)
          + tuple(pltpu.SemaphoreType.DMA for _ in range(4)),
    )
    def fuse(idx_hbm, vert_hbm, z_hbm, sd_hbm, out_hbm,
             idx_v, vert_v, sd_v, idx2_v,
             rows0_v, rows1_v,
             outs0_v, outs1_v, z_sh,
             gsem0, gsem1, osem0, osem1):
        c = lax.axis_index("c")
        s = lax.axis_index("s")
        trow = (c * _NS + s) * CHN
        pltpu.sync_copy(idx_hbm.at[pl.ds(trow, CHN)], idx_v)
        pltpu.sync_copy(vert_hbm.at[pl.ds(c * V * 3, V * 3)],
                        vert_v.at[pl.ds(0, V * 3)])
        pltpu.sync_copy(sd_hbm, sd_v)

        pltpu.sync_copy(z_hbm.at[pl.ds(c * VH + s * ZR, ZR)],
                        z_sh.at[pl.ds(s * ZR, ZR)])

        @pl.when(s == 0)
        def _():
            rem = VH - _NS * ZR
            pltpu.sync_copy(z_hbm.at[pl.ds(c * VH + _NS * ZR, rem)],
                            z_sh.at[pl.ds(_NS * ZR, rem)])

        def shift_body(q, carry):
            row = q // (CHW // _L)
            col = (q % (CHW // _L)) * _L
            idx2_v[row, pl.ds(col, _L)] = lax.shift_right_logical(
                idx_v[row, pl.ds(col, _L)], 1)
            return carry

        lax.fori_loop(0, CHN * (CHW // _L), shift_body, 0)

        plsc.subcore_barrier()

        sdv = [[sd_v[r, pl.ds(cg * _L, _L)] for cg in range(CG)]
               for r in range(4)]
        rows = (rows0_v, rows1_v)
        outs = (outs0_v, outs1_v)
        gsems = (gsem0, gsem1)
        osems = (osem0, osem1)
        NB = 2
        NP = CHN // NB

        def fire_gather(j, buf):
            pltpu.async_copy(z_sh.at[idx2_v.at[j]], rows[buf], gsems[buf])

        def wait_gather(buf):
            pltpu.make_async_copy(z_sh.at[idx2_v.at[0]], rows[buf],
                                  gsems[buf]).wait()

        def out_slot(j):
            return out_hbm.at[pl.ds(c * VP + s * VPT + j * VPC, VPC)]

        def compute_chunk(j, buf, m):
            wait_gather(buf)
            @pl.when(m > 0)
            def _():
                pltpu.make_async_copy(outs[buf], out_slot(0),
                                      osems[buf]).wait()
            rv = rows[buf]
            ov = outs[buf]

            def vert_body(k, carry):
                vg = s * VPT + j * VPC + k
                i16 = idx_v[j, pl.ds(k * N, N)]
                i3 = i16 * 3
                xg = plsc.load_gather(vert_v, [i3])
                yg = plsc.load_gather(vert_v, [i3 + 1])
                zg = plsc.load_gather(vert_v, [i3 + 2])
                dx = xg - _splat(vert_v, [3 * vg])
                dy = yg - _splat(vert_v, [3 * vg + 1])
                dz = zg - _splat(vert_v, [3 * vg + 2])
                r = _rsqrt16(dx * dx + dy * dy + dz * dz)
                n1v = (dx * r + 1.0) * 0.5
                n2v = (dy * r + 1.0) * 0.5
                n3v = (dz * r + 1.0) * 0.5
                accs = [jnp.zeros((_L,), jnp.float32) for _ in range(CG)]
                for i in range(N):
                    n1 = lax.broadcast(n1v[i], (_L,))
                    n2 = lax.broadcast(n2v[i], (_L,))
                    n3 = lax.broadcast(n3v[i], (_L,))
                    uoff = (i16[i] & 1) * (OC // 2)
                    for h in range(CG // 2):
                        zb = rv[k * N + i, pl.ds(uoff + h * _L, _L)]
                        zu = plsc.unpack(
                            plsc.bitcast(zb, jnp.bfloat16),
                            format=plsc.PackFormat.INTERLEAVED)
                        for half in range(2):
                            cg = 2 * h + half
                            w = (sdv[3][cg] + n1 * sdv[0][cg]
                                 + n2 * sdv[1][cg] + n3 * sdv[2][cg])
                            accs[cg] = jnp.maximum(accs[cg],
                                                   w * zu[half])
                for cg in range(CG):
                    li = ((cg // 2) * 2 * _L + (cg % 2)
                          + 2 * lax.iota(jnp.int32, _L))
                    plsc.store_scatter(
                        ov, [jnp.full((_L,), k, jnp.int32), li], accs[cg])
                return carry

            lax.fori_loop(0, VPC, vert_body, 0)
            pltpu.async_copy(ov, out_slot(j), osems[buf])

            @pl.when(m < NP - 1)
            def _():
                fire_gather(j + NB, buf)

        for b in range(NB):
            fire_gather(b, b)

        def group_body(m, carry):
            for b in range(NB):
                compute_chunk(NB * m + b, b, m)
            return carry

        lax.fori_loop(0, NP, group_body, 0)
        for b in range(NB):
            pltpu.make_async_copy(outs[b], out_slot(0), osems[b]).wait()

    return fuse


def _p_body(x_ref, wt_ref, b_ref, p_ref):
    x = x_ref[...]
    nrm = jnp.sqrt(jnp.sum(x * x, axis=1, keepdims=True))
    lane = lax.broadcasted_iota(jnp.int32, x.shape, 1)
    xa = jnp.where(lane == x.shape[1] - 1, nrm, x)
    p_ref[...] = (jnp.dot(xa, wt_ref[...], preferred_element_type=jnp.float32,
                          precision=lax.Precision.HIGHEST) + b_ref[...])


def _make_stats_body(nrows):
    def stats_body(p_ref, cnt_ref, g_ref, be_ref, ad_ref, acc_ref):
        step = pl.program_id(0)

        @pl.when(step == 0)
        def _():
            acc_ref[...] = jnp.zeros_like(acc_ref)

        p = p_ref[...]
        cp = cnt_ref[...] * p
        s = jnp.sum(cp, axis=0, keepdims=True)
        q = jnp.sum(cp * p, axis=0, keepdims=True)
        acc_ref[...] += jnp.concatenate([s, q], axis=0)

        @pl.when(step == pl.num_programs(0) - 1)
        def _():
            acc = acc_ref[...]
            mean = acc[0:1] / nrows
            var = acc[1:2] / nrows - mean * mean
            a = g_ref[...] * lax.rsqrt(var + 1e-5)
            d = be_ref[...] - mean * a
            ad_ref[...] = jnp.concatenate([a, d], axis=0)

    return stats_body


def _z_body(p_ref, ad_ref, z_ref):
    ad = ad_ref[...]
    z_ref[...] = jnp.maximum(p_ref[...] * ad[0:1] + ad[1:2],
                             0.0).astype(jnp.bfloat16)


def kernel(neighbor_index, vertices, feature_map, directions, W, b, gamma,
           beta):
    BS, V, N = neighbor_index.shape
    OC, IC = W.shape
    NR = BS * V * N
    CHW = 8 * N
    VP = 10240
    CHN = VP * N // _NS // CHW

    pad = jnp.full((BS, VP - V, N), V, jnp.int32)
    idx_loc = jnp.concatenate([neighbor_index, pad], axis=1)
    idx_loc = idx_loc.reshape(BS * _NS * CHN, CHW)
    vert_flat = vertices.reshape(BS * V * 3)
    fm_pad = jnp.pad(feature_map.reshape(BS * V, IC - 1), ((0, 0), (0, 1)))
    wt = W.T
    brow = b.reshape(1, OC)
    sd4 = jnp.stack([directions[1] - directions[0],
                     directions[2] - directions[0],
                     directions[3] - directions[0],
                     directions[0]], axis=0)
    perm = jnp.asarray(
        [(cg // 2) * 32 + 2 * lane + (cg % 2)
         for cg in range(OC // _L) for lane in range(_L)], jnp.int32)
    sd = jnp.concatenate(
        [sd4[:, perm], jnp.zeros((4, OC), jnp.float32)], axis=0)
    zeros_v = jnp.zeros((VP,), jnp.float32)
    ones_c = jnp.ones((CHW,), jnp.float32)

    counts = _make_prep(BS, V, VP, N, CHN, CHW)(idx_loc, zeros_v, ones_c)
    counts = counts.reshape(BS, VP)[:, :V].reshape(BS * V, 1)

    RB = 2000
    NGB = BS * V // RB
    P = pl.pallas_call(
        _p_body,
        grid=(NGB,),
        in_specs=[
            pl.BlockSpec((RB, IC), lambda i: (i, 0)),
            pl.BlockSpec((IC, OC), lambda i: (0, 0)),
            pl.BlockSpec((1, OC), lambda i: (0, 0)),
        ],
        out_specs=pl.BlockSpec((RB, OC), lambda i: (i, 0)),
        out_shape=jax.ShapeDtypeStruct((BS * V, OC), jnp.float32),
    )(fm_pad, wt, brow)

    ad = pl.pallas_call(
        _make_stats_body(float(NR)),
        grid=(NGB,),
        in_specs=[
            pl.BlockSpec((RB, OC), lambda i: (i, 0)),
            pl.BlockSpec((RB, 1), lambda i: (i, 0)),
            pl.BlockSpec((1, OC), lambda i: (0, 0)),
            pl.BlockSpec((1, OC), lambda i: (0, 0)),
        ],
        out_specs=pl.BlockSpec((2, OC), lambda i: (0, 0)),
        out_shape=jax.ShapeDtypeStruct((2, OC), jnp.float32),
        scratch_shapes=[pltpu.VMEM((2, OC), jnp.float32)],
    )(P, counts, gamma.reshape(1, OC), beta.reshape(1, OC))

    Z = pl.pallas_call(
        _z_body,
        grid=(NGB,),
        in_specs=[
            pl.BlockSpec((RB, OC), lambda i: (i, 0)),
            pl.BlockSpec((2, OC), lambda i: (0, 0)),
        ],
        out_specs=pl.BlockSpec((RB, OC), lambda i: (i, 0)),
        out_shape=jax.ShapeDtypeStruct((BS * V, OC), jnp.bfloat16),
    )(P, ad)

    z_pairs = lax.bitcast_convert_type(
        Z.reshape(BS * V // 2, OC, 2), jnp.int32)
    out = _make_fuse(BS, V, VP, N, OC, CHN, CHW)(idx_loc, vert_flat,
                                                 z_pairs, sd)
    return out.reshape(BS, VP, OC)[:, :V, :]


# speedup vs baseline: 38.2680x; 3.8651x over previous
import functools

import jax
import jax.numpy as jnp
from jax import lax
from jax.experimental import pallas as pl
from jax.experimental.pallas import tpu as pltpu
from jax.experimental.pallas import tpu_sc as plsc

_NC = 2
_NS = 16
_L = 16


def _rsqrt16(nsq):
    i = plsc.bitcast(nsq, jnp.int32)
    i = jnp.int32(0x5F3759DF) - lax.shift_right_logical(i, 1)
    r = plsc.bitcast(i, jnp.float32)
    half = nsq * 0.5
    for _ in range(3):
        r = r * (1.5 - half * r * r)
    return jnp.minimum(r, 1e12)


def _splat(ref, idxs):
    return plsc.load_gather(
        ref, [jnp.full((_L,), i, jnp.int32) for i in idxs])


def _make_prep(BS, V, VP, N, CHN, CHW):
    mesh = plsc.VectorSubcoreMesh(core_axis_name="c", subcore_axis_name="s")

    @functools.partial(
        pl.kernel,
        out_type=jax.ShapeDtypeStruct((BS * VP,), jnp.float32),
        mesh=mesh,
        compiler_params=pltpu.CompilerParams(needs_layout_passes=False),
        scratch_types=(
            pltpu.VMEM((CHN, CHW), jnp.int32),
            pltpu.VMEM((CHW,), jnp.float32),
            pltpu.VMEM_SHARED((VP,), jnp.float32),
            pltpu.SemaphoreType.DMA,
        ),
    )
    def prep(idx_hbm, zeros_hbm, ones_hbm, cnt_hbm,
             idx_v, ones_v, hist_sh, sem):
        c = lax.axis_index("c")
        s = lax.axis_index("s")
        trow = (c * _NS + s) * CHN
        pltpu.sync_copy(idx_hbm.at[pl.ds(trow, CHN)], idx_v)
        pltpu.sync_copy(ones_hbm, ones_v)

        @pl.when(s == 0)
        def _():
            pltpu.sync_copy(zeros_hbm, hist_sh)

        plsc.subcore_barrier()

        def chunk_body(j, carry):
            pltpu.sync_copy(ones_v, hist_sh.at[idx_v.at[j]], add=True)
            return carry

        lax.fori_loop(0, CHN, chunk_body, 0)

        plsc.subcore_barrier()

        @pl.when(s == 0)
        def _():
            pltpu.sync_copy(hist_sh, cnt_hbm.at[pl.ds(c * VP, VP)])

    return prep


def _make_fuse(BS, V, VP, N, OC, CHN, CHW):
    VPC = CHW // N
    VPT = VP // _NS
    CG = OC // _L
    VH = V // 2
    ZR = (VH // _NS) // 8 * 8
    VZ = VH + 8
    mesh = plsc.VectorSubcoreMesh(core_axis_name="c", subcore_axis_name="s")

    @functools.partial(
        pl.kernel,
        out_type=jax.ShapeDtypeStruct((BS * VP, OC), jnp.float32),
        mesh=mesh,
        compiler_params=pltpu.CompilerParams(needs_layout_passes=False),
        scratch_types=(
            pltpu.VMEM((CHN, CHW), jnp.int32),
            pltpu.VMEM((3 * VP,), jnp.float32),
            pltpu.VMEM((8, OC), jnp.float32),
        ) + (pltpu.VMEM((CHN, CHW), jnp.int32),)
          + tuple(pltpu.VMEM((CHW, OC), jnp.int32) for _ in range(2))
          + tuple(pltpu.VMEM((VPC, OC), jnp.float32) for _ in range(2))
          + (pltpu.VMEM_SHARED((VZ, OC), jnp.int32),)
          + tuple(pltpu.SemaphoreType.DMA for _ in range(4)),
    )
    def fuse(idx_hbm, vert_hbm, z_hbm, sd_hbm, out_hbm,
             idx_v, vert_v, sd_v, idx2_v,
             rows0_v, rows1_v,
             outs0_v, outs1_v, z_sh,
             gsem0, gsem1, osem0, osem1):
        c = lax.axis_index("c")
        s = lax.axis_index("s")
        trow = (c * _NS + s) * CHN
        pltpu.sync_copy(idx_hbm.at[pl.ds(trow, CHN)], idx_v)
        pltpu.sync_copy(vert_hbm.at[pl.ds(c * V * 3, V * 3)],
                        vert_v.at[pl.ds(0, V * 3)])
        pltpu.sync_copy(sd_hbm, sd_v)

        pltpu.sync_copy(z_hbm.at[pl.ds(c * VH + s * ZR, ZR)],
                        z_sh.at[pl.ds(s * ZR, ZR)])

        @pl.when(s == 0)
        def _():
            rem = VH - _NS * ZR
            pltpu.sync_copy(z_hbm.at[pl.ds(c * VH + _NS * ZR, rem)],
                            z_sh.at[pl.ds(_NS * ZR, rem)])

        def shift_body(q, carry):
            row = q // (CHW // _L)
            col = (q % (CHW // _L)) * _L
            v = idx_v[row, pl.ds(col, _L)]
            idx2_v[row, pl.ds(col, _L)] = jnp.where(v >= VH, v - VH, v)
            return carry

        lax.fori_loop(0, CHN * (CHW // _L), shift_body, 0)

        plsc.subcore_barrier()

        sdv = [[sd_v[r, pl.ds(cg * _L, _L)] for cg in range(CG)]
               for r in range(4)]
        rows = (rows0_v, rows1_v)
        outs = (outs0_v, outs1_v)
        gsems = (gsem0, gsem1)
        osems = (osem0, osem1)
        NB = 2
        NP = CHN // NB

        def fire_gather(j, buf):
            pltpu.async_copy(z_sh.at[idx2_v.at[j]], rows[buf], gsems[buf])

        def wait_gather(buf):
            pltpu.make_async_copy(z_sh.at[idx2_v.at[0]], rows[buf],
                                  gsems[buf]).wait()

        def out_slot(j):
            return out_hbm.at[pl.ds(c * VP + s * VPT + j * VPC, VPC)]

        def compute_chunk(j, buf, m):
            wait_gather(buf)
            @pl.when(m > 0)
            def _():
                pltpu.make_async_copy(outs[buf], out_slot(0),
                                      osems[buf]).wait()
            rv = rows[buf]
            ov = outs[buf]

            def vert_body(k, carry):
                vg = s * VPT + j * VPC + k
                i16 = idx_v[j, pl.ds(k * N, N)]
                i3 = i16 * 3
                xg = plsc.load_gather(vert_v, [i3])
                yg = plsc.load_gather(vert_v, [i3 + 1])
                zg = plsc.load_gather(vert_v, [i3 + 2])
                dx = xg - _splat(vert_v, [3 * vg])
                dy = yg - _splat(vert_v, [3 * vg + 1])
                dz = zg - _splat(vert_v, [3 * vg + 2])
                r = _rsqrt16(dx * dx + dy * dy + dz * dz)
                n1v = (dx * r + 1.0) * 0.5
                n2v = (dy * r + 1.0) * 0.5
                n3v = (dz * r + 1.0) * 0.5
                accs = [jnp.zeros((_L,), jnp.float32) for _ in range(CG)]
                for i in range(N):
                    n1 = lax.broadcast(n1v[i], (_L,))
                    n2 = lax.broadcast(n2v[i], (_L,))
                    n3 = lax.broadcast(n3v[i], (_L,))
                    uoff = jnp.where(i16[i] >= VH, OC // 2, 0)
                    for h in range(CG // 2):
                        zb = rv[k * N + i, pl.ds(uoff + h * _L, _L)]
                        zu = plsc.unpack(
                            plsc.bitcast(zb, jnp.bfloat16),
                            format=plsc.PackFormat.INTERLEAVED)
                        for half in range(2):
                            cg = h + 4 * half
                            w = (sdv[3][cg] + n1 * sdv[0][cg]
                                 + n2 * sdv[1][cg] + n3 * sdv[2][cg])
                            accs[cg] = jnp.maximum(accs[cg],
                                                   w * zu[half])
                for cg in range(CG):
                    ov[k, pl.ds(cg * _L, _L)] = accs[cg]
                return carry

            lax.fori_loop(0, VPC, vert_body, 0)
            pltpu.async_copy(ov, out_slot(j), osems[buf])

            @pl.when(m < NP - 1)
            def _():
                fire_gather(j + NB, buf)

        for b in range(NB):
            fire_gather(b, b)

        def group_body(m, carry):
            for b in range(NB):
                compute_chunk(NB * m + b, b, m)
            return carry

        lax.fori_loop(0, NP, group_body, 0)
        for b in range(NB):
            pltpu.make_async_copy(outs[b], out_slot(0), osems[b]).wait()

    return fuse


def _p_body(x_ref, wt_ref, b_ref, p_ref):
    x = x_ref[...]
    nrm = jnp.sqrt(jnp.sum(x * x, axis=1, keepdims=True))
    lane = lax.broadcasted_iota(jnp.int32, x.shape, 1)
    xa = jnp.where(lane == x.shape[1] - 1, nrm, x)
    p_ref[...] = (jnp.dot(xa, wt_ref[...], preferred_element_type=jnp.float32,
                          precision=lax.Precision.HIGHEST) + b_ref[...])


def _make_stats_body(nrows):
    def stats_body(p_ref, cnt_ref, g_ref, be_ref, ad_ref, acc_ref):
        step = pl.program_id(0)

        @pl.when(step == 0)
        def _():
            acc_ref[...] = jnp.zeros_like(acc_ref)

        p = p_ref[...]
        cp = cnt_ref[...] * p
        s = jnp.sum(cp, axis=0, keepdims=True)
        q = jnp.sum(cp * p, axis=0, keepdims=True)
        acc_ref[...] += jnp.concatenate([s, q], axis=0)

        @pl.when(step == pl.num_programs(0) - 1)
        def _():
            acc = acc_ref[...]
            mean = acc[0:1] / nrows
            var = acc[1:2] / nrows - mean * mean
            a = g_ref[...] * lax.rsqrt(var + 1e-5)
            d = be_ref[...] - mean * a
            ad_ref[...] = jnp.concatenate([a, d], axis=0)

    return stats_body


def _pack_half(z):
    h = z.shape[1] // 2
    lo = lax.bitcast_convert_type(z[:, :h].astype(jnp.bfloat16),
                                  jnp.uint16).astype(jnp.uint32)
    hi = lax.bitcast_convert_type(z[:, h:].astype(jnp.bfloat16),
                                  jnp.uint16).astype(jnp.uint32)
    return lax.bitcast_convert_type(lo | (hi << 16), jnp.int32)


def _z_body(pa_ref, pb_ref, ad_ref, z_ref):
    ad = ad_ref[...]
    za = jnp.maximum(pa_ref[...] * ad[0:1] + ad[1:2], 0.0)
    zb = jnp.maximum(pb_ref[...] * ad[0:1] + ad[1:2], 0.0)
    z_ref[...] = jnp.concatenate([_pack_half(za), _pack_half(zb)], axis=1)


def kernel(neighbor_index, vertices, feature_map, directions, W, b, gamma,
           beta):
    BS, V, N = neighbor_index.shape
    OC, IC = W.shape
    NR = BS * V * N
    CHW = 8 * N
    VP = 10240
    CHN = VP * N // _NS // CHW

    pad = jnp.full((BS, VP - V, N), V, jnp.int32)
    idx_loc = jnp.concatenate([neighbor_index, pad], axis=1)
    idx_loc = idx_loc.reshape(BS * _NS * CHN, CHW)
    vert_flat = vertices.reshape(BS * V * 3)
    fm_pad = jnp.pad(feature_map.reshape(BS * V, IC - 1), ((0, 0), (0, 1)))
    wt = W.T
    brow = b.reshape(1, OC)
    sd = jnp.concatenate(
        [jnp.stack([directions[1] - directions[0],
                    directions[2] - directions[0],
                    directions[3] - directions[0],
                    directions[0]], axis=0),
         jnp.zeros((4, OC), jnp.float32)], axis=0)
    zeros_v = jnp.zeros((VP,), jnp.float32)
    ones_c = jnp.ones((CHW,), jnp.float32)

    counts = _make_prep(BS, V, VP, N, CHN, CHW)(idx_loc, zeros_v, ones_c)
    counts = counts.reshape(BS, VP)[:, :V].reshape(BS * V, 1)

    RB = 2000
    NGB = BS * V // RB
    P = pl.pallas_call(
        _p_body,
        grid=(NGB,),
        in_specs=[
            pl.BlockSpec((RB, IC), lambda i: (i, 0)),
            pl.BlockSpec((IC, OC), lambda i: (0, 0)),
            pl.BlockSpec((1, OC), lambda i: (0, 0)),
        ],
        out_specs=pl.BlockSpec((RB, OC), lambda i: (i, 0)),
        out_shape=jax.ShapeDtypeStruct((BS * V, OC), jnp.float32),
    )(fm_pad, wt, brow)

    ad = pl.pallas_call(
        _make_stats_body(float(NR)),
        grid=(NGB,),
        in_specs=[
            pl.BlockSpec((RB, OC), lambda i: (i, 0)),
            pl.BlockSpec((RB, 1), lambda i: (i, 0)),
            pl.BlockSpec((1, OC), lambda i: (0, 0)),
            pl.BlockSpec((1, OC), lambda i: (0, 0)),
        ],
        out_specs=pl.BlockSpec((2, OC), lambda i: (0, 0)),
        out_shape=jax.ShapeDtypeStruct((2, OC), jnp.float32),
        scratch_shapes=[pltpu.VMEM((2, OC), jnp.float32)],
    )(P, counts, gamma.reshape(1, OC), beta.reshape(1, OC))

    VH = V // 2
    RB2 = 1000
    NH = VH // RB2
    z_pairs = pl.pallas_call(
        _z_body,
        grid=(BS * NH,),
        in_specs=[
            pl.BlockSpec((RB2, OC), lambda i: (i + (i // NH) * NH, 0)),
            pl.BlockSpec((RB2, OC),
                         lambda i: (i + (i // NH) * NH + NH, 0)),
            pl.BlockSpec((2, OC), lambda i: (0, 0)),
        ],
        out_specs=pl.BlockSpec((RB2, OC), lambda i: (i, 0)),
        out_shape=jax.ShapeDtypeStruct((BS * VH, OC), jnp.int32),
    )(P, P, ad)

    out = _make_fuse(BS, V, VP, N, OC, CHN, CHW)(idx_loc, vert_flat,
                                                 z_pairs, sd)
    return out.reshape(BS, VP, OC)[:, :V, :]
